# Initial kernel scaffold; baseline (speedup 1.0000x reference)
import jax
import jax.numpy as jnp
from jax.experimental import pallas as pl


def kernel(x, edge_index, edge_weight, W1, b1, W2, b2):
    raise NotImplementedError("write your pallas kernel here")


# speedup vs baseline: 16.6612x; 16.6612x over previous
import jax
import jax.numpy as jnp
from jax import lax
from jax.experimental import pallas as pl
from jax.experimental.pallas import tpu as pltpu
from jax.experimental.pallas import tpu_sc as plsc
import functools

NC = 2
NS = 16
NW = NC * NS
K = 80


def _wid():
    c = lax.axis_index("c")
    s = lax.axis_index("s")
    return s * NC + c, c, s


def _make_deg_kernel(n, nch):
    mesh = plsc.VectorSubcoreMesh(core_axis_name="c", subcore_axis_name="s")

    @functools.partial(
        pl.kernel,
        out_type=jax.ShapeDtypeStruct((NC, n), jnp.float32),
        mesh=mesh,
        compiler_params=pltpu.CompilerParams(use_tc_tiling_on_sc=False, needs_layout_passes=False),
        scratch_types=[
            pltpu.VMEM((nch, K), jnp.int32),
            pltpu.VMEM((nch, K), jnp.float32),
            pltpu.VMEM((n,), jnp.float32),
            pltpu.VMEM_SHARED((n,), jnp.float32),
        ],
    )
    def deg_kernel(dst_hbm, ew_hbm, out_hbm, dst_v, ew_v, tmp_v, acc_sh):
        wid, c, s = _wid()
        pltpu.sync_copy(dst_hbm.at[wid], dst_v)
        pltpu.sync_copy(ew_hbm.at[wid], ew_v)

        @pl.when(s == 0)
        def _zero():
            def zrow(i, _):
                tmp_v[pl.ds(i * 16, 16)] = jnp.zeros((16,), jnp.float32)
                return 0
            lax.fori_loop(0, n // 16, zrow, 0)
            pltpu.sync_copy(tmp_v, acc_sh)

        plsc.subcore_barrier()

        def body(j, _):
            pltpu.sync_copy(ew_v.at[j], acc_sh.at[dst_v.at[j]], add=True)
            return 0
        lax.fori_loop(0, nch, body, 0)

        plsc.subcore_barrier()

        @pl.when(s == 0)
        def _writeout():
            pltpu.sync_copy(acc_sh, tmp_v)
            pltpu.sync_copy(tmp_v, out_hbm.at[c])

    return deg_kernel


def _make_msg_kernel(n, nch, d):
    mesh = plsc.VectorSubcoreMesh(core_axis_name="c", subcore_axis_name="s")
    rows_per_sub = n // NS
    stage_rows = rows_per_sub // 25
    nblk = rows_per_sub // stage_rows
    nfc = d // 16

    @functools.partial(
        pl.kernel,
        out_type=jax.ShapeDtypeStruct((NC, n, d), jnp.float32),
        mesh=mesh,
        compiler_params=pltpu.CompilerParams(use_tc_tiling_on_sc=False, needs_layout_passes=False),
        scratch_types=[
            pltpu.VMEM((nch, K), jnp.int32),
            pltpu.VMEM((nch, K), jnp.int32),
            pltpu.VMEM((nch, K), jnp.float32),
            pltpu.VMEM((K, d), jnp.float32),
            pltpu.VMEM((stage_rows, d), jnp.float32),
            pltpu.VMEM_SHARED((n, d), jnp.float32),
        ],
    )
    def msg_kernel(g_hbm, src_hbm, dst_hbm, ew_hbm, out_hbm,
                   src_v, dst_v, ew_v, rows_v, stage_v, acc_sh):
        wid, c, s = _wid()
        pltpu.sync_copy(src_hbm.at[wid], src_v)
        pltpu.sync_copy(dst_hbm.at[wid], dst_v)
        pltpu.sync_copy(ew_hbm.at[wid], ew_v)

        base = s * rows_per_sub

        def init_blk(k2, _):
            off = base + k2 * stage_rows
            pltpu.sync_copy(g_hbm.at[pl.ds(off, stage_rows)], stage_v)
            pltpu.sync_copy(stage_v, acc_sh.at[pl.ds(off, stage_rows)])
            return 0
        lax.fori_loop(0, nblk, init_blk, 0)

        plsc.subcore_barrier()

        def chunk(j, _):
            pltpu.sync_copy(g_hbm.at[src_v.at[j]], rows_v)

            def scale_grp(g2, _):
                for l in range(16):
                    i = g2 * 16 + l
                    spl = plsc.load_gather(
                        ew_v,
                        [jnp.full((16,), j, jnp.int32),
                         jnp.full((16,), i, jnp.int32)],
                    )
                    for fc in range(nfc):
                        rows_v[i, pl.ds(fc * 16, 16)] = (
                            rows_v[i, pl.ds(fc * 16, 16)] * spl)
                return 0
            lax.fori_loop(0, K // 16, scale_grp, 0)

            pltpu.sync_copy(rows_v, acc_sh.at[dst_v.at[j]], add=True)
            return 0
        lax.fori_loop(0, nch, chunk, 0)

        plsc.subcore_barrier()

        def out_blk(k2, _):
            off = base + k2 * stage_rows
            pltpu.sync_copy(acc_sh.at[pl.ds(off, stage_rows)], stage_v)
            pltpu.sync_copy(stage_v, out_hbm.at[c].at[pl.ds(off, stage_rows)])
            return 0
        lax.fori_loop(0, nblk, out_blk, 0)

    return msg_kernel


def _tc_call(body, out_shapes, inputs):
    return pl.pallas_call(body, out_shape=out_shapes)(*inputs)


def _dinv_from_parts(dp0, dp1):
    deg = 1.0 + dp0 + dp1
    return jnp.where(deg > 0, lax.rsqrt(deg), 0.0)


def _tc0_body(x_ref, w1_ref, h_ref):
    h_ref[...] = jnp.dot(x_ref[...], w1_ref[...],
                         preferred_element_type=jnp.float32)


def _tc1_body(h_ref, dp_ref, g1_ref):
    dinv = _dinv_from_parts(dp_ref[0, :], dp_ref[1, :])
    g1_ref[...] = h_ref[...] * dinv[:, None]


def _tc2_body(acc_ref, g1_ref, dp_ref, w2_ref, b1_ref, g2_ref):
    dinv = _dinv_from_parts(dp_ref[0, :], dp_ref[1, :])
    t = acc_ref[0] + acc_ref[1] - g1_ref[...]
    pre = t * dinv[:, None] + b1_ref[...][None, :]
    r = jnp.maximum(pre, 0.0)
    h2 = jnp.dot(r, w2_ref[...], preferred_element_type=jnp.float32)
    g2_ref[...] = h2 * dinv[:, None]


def _tc3_body(acc_ref, g2_ref, dp_ref, b2_ref, out_ref):
    dinv = _dinv_from_parts(dp_ref[0, :], dp_ref[1, :])
    t = acc_ref[0] + acc_ref[1] - g2_ref[...]
    o = t * dinv[:, None] + b2_ref[...][None, :]
    m = jnp.max(o, axis=1, keepdims=True)
    e = jnp.exp(o - m)
    lse = jnp.log(jnp.sum(e, axis=1, keepdims=True))
    out_ref[...] = o - m - lse


def kernel(x, edge_index, edge_weight, W1, b1, W2, b2):
    n, d_in = x.shape
    e = edge_weight.shape[0]
    d_hid = W1.shape[1]
    d_out = W2.shape[1]
    epw = e // NW
    nch = epw // K

    src3 = edge_index[0].reshape(NW, nch, K)
    dst3 = edge_index[1].reshape(NW, nch, K)
    ew3 = edge_weight.reshape(NW, nch, K)

    h = _tc_call(_tc0_body, jax.ShapeDtypeStruct((n, d_hid), jnp.float32),
                 [x, W1])
    degp = _make_deg_kernel(n, nch)(dst3, ew3)
    g1 = _tc_call(_tc1_body, jax.ShapeDtypeStruct((n, d_hid), jnp.float32),
                  [h, degp])
    acc1 = _make_msg_kernel(n, nch, d_hid)(g1, src3, dst3, ew3)
    g2 = _tc_call(_tc2_body, jax.ShapeDtypeStruct((n, d_out), jnp.float32),
                  [acc1, g1, degp, W2, b1])
    acc2 = _make_msg_kernel(n, nch, d_out)(g2, src3, dst3, ew3)
    out = _tc_call(_tc3_body, jax.ShapeDtypeStruct((n, d_out), jnp.float32),
                   [acc2, g2, degp, b2])
    return out


# speedup vs baseline: 27.1780x; 1.6312x over previous
import jax
import jax.numpy as jnp
from jax import lax
from jax.experimental import pallas as pl
from jax.experimental.pallas import tpu as pltpu
from jax.experimental.pallas import tpu_sc as plsc
import functools

NC = 2
NS = 16
NW = NC * NS
K = 80


def _wid():
    c = lax.axis_index("c")
    s = lax.axis_index("s")
    return s * NC + c, c, s


def _make_deg_kernel(n, nch):
    mesh = plsc.VectorSubcoreMesh(core_axis_name="c", subcore_axis_name="s")

    @functools.partial(
        pl.kernel,
        out_type=jax.ShapeDtypeStruct((NC, n), jnp.float32),
        mesh=mesh,
        compiler_params=pltpu.CompilerParams(use_tc_tiling_on_sc=False, needs_layout_passes=False),
        scratch_types=[
            pltpu.VMEM((nch, K), jnp.int32),
            pltpu.VMEM((nch, K), jnp.float32),
            pltpu.VMEM((n,), jnp.float32),
            pltpu.VMEM_SHARED((n,), jnp.float32),
        ],
    )
    def deg_kernel(dst_hbm, ew_hbm, out_hbm, dst_v, ew_v, tmp_v, acc_sh):
        wid, c, s = _wid()
        pltpu.sync_copy(dst_hbm.at[wid], dst_v)
        pltpu.sync_copy(ew_hbm.at[wid], ew_v)

        @pl.when(s == 0)
        def _zero():
            def zrow(i, _):
                tmp_v[pl.ds(i * 16, 16)] = jnp.zeros((16,), jnp.float32)
                return 0
            lax.fori_loop(0, n // 16, zrow, 0)
            pltpu.sync_copy(tmp_v, acc_sh)

        plsc.subcore_barrier()

        def body(j, _):
            pltpu.sync_copy(ew_v.at[j], acc_sh.at[dst_v.at[j]], add=True)
            return 0
        lax.fori_loop(0, nch, body, 0)

        plsc.subcore_barrier()

        @pl.when(s == 0)
        def _writeout():
            pltpu.sync_copy(acc_sh, tmp_v)
            pltpu.sync_copy(tmp_v, out_hbm.at[c])

    return deg_kernel


def _make_msg_kernel(n, nch, d, nbuf, prologue):
    mesh = plsc.VectorSubcoreMesh(core_axis_name="c", subcore_axis_name="s")
    rows_per_sub = n // NS
    m = (nch - prologue) // nbuf
    nfc = d // 16

    @functools.partial(
        pl.kernel,
        out_type=jax.ShapeDtypeStruct((NC, n, d), jnp.float32),
        mesh=mesh,
        compiler_params=pltpu.CompilerParams(use_tc_tiling_on_sc=False, needs_layout_passes=False),
        scratch_types=[
            pltpu.VMEM((nch, K), jnp.int32),
            pltpu.VMEM((nch, K), jnp.int32),
            pltpu.VMEM((nch, K), jnp.float32),
        ] + [pltpu.VMEM((K, d), jnp.float32) for _ in range(nbuf)]
          + [pltpu.SemaphoreType.DMA for _ in range(nbuf)]
          + [pltpu.VMEM_SHARED((n, d), jnp.float32)],
    )
    def msg_kernel(g_hbm, src_hbm, dst_hbm, ew_hbm, out_hbm,
                   src_v, dst_v, ew_v, *rest):
        rows = rest[:nbuf]
        sems = rest[nbuf:2 * nbuf]
        acc_sh = rest[2 * nbuf]
        wid, c, s = _wid()
        pltpu.sync_copy(src_hbm.at[wid], src_v)
        pltpu.sync_copy(dst_hbm.at[wid], dst_v)
        pltpu.sync_copy(ew_hbm.at[wid], ew_v)

        base = s * rows_per_sub
        pltpu.sync_copy(g_hbm.at[pl.ds(base, rows_per_sub)],
                        acc_sh.at[pl.ds(base, rows_per_sub)])

        plsc.subcore_barrier()

        def scale(j, rv):
            def scale_grp(g2, _):
                for l in range(16):
                    i = g2 * 16 + l
                    spl = plsc.load_gather(
                        ew_v,
                        [jnp.full((16,), j, jnp.int32),
                         jnp.full((16,), i, jnp.int32)],
                    )
                    for fc in range(nfc):
                        rv[i, pl.ds(fc * 16, 16)] = (
                            rv[i, pl.ds(fc * 16, 16)] * spl)
                return 0
            lax.fori_loop(0, K // 16, scale_grp, 0)

        def start(j, b):
            pltpu.make_async_copy(g_hbm.at[src_v.at[j]], rows[b],
                                  sems[b]).start()

        def wait(j, b):
            pltpu.make_async_copy(g_hbm.at[src_v.at[j]], rows[b],
                                  sems[b]).wait()

        def scatter(j, b):
            pltpu.sync_copy(rows[b], acc_sh.at[dst_v.at[j]], add=True)

        for p in range(prologue):
            pltpu.sync_copy(g_hbm.at[src_v.at[p]], rows[0])
            scale(p, rows[0])
            scatter(p, 0)

        for b in range(nbuf):
            start(prologue + b, b)

        def grp(g, _):
            for b in range(nbuf):
                j = prologue + g * nbuf + b
                wait(j, b)
                scale(j, rows[b])
                scatter(j, b)
                start(j + nbuf, b)
            return 0
        lax.fori_loop(0, m - 1, grp, 0)

        for b in range(nbuf):
            j = prologue + (m - 1) * nbuf + b
            wait(j, b)
            scale(j, rows[b])
            scatter(j, b)

        plsc.subcore_barrier()

        pltpu.sync_copy(acc_sh.at[pl.ds(base, rows_per_sub)],
                        out_hbm.at[c].at[pl.ds(base, rows_per_sub)])

    return msg_kernel


def _tc_call(body, out_shapes, inputs):
    return pl.pallas_call(body, out_shape=out_shapes)(*inputs)


def _dinv_from_parts(dp0, dp1):
    deg = 1.0 + dp0 + dp1
    return jnp.where(deg > 0, lax.rsqrt(deg), 0.0)


def _tc0_body(x_ref, w1_ref, h_ref):
    h_ref[...] = jnp.dot(x_ref[...], w1_ref[...],
                         preferred_element_type=jnp.float32)


def _tc1_body(h_ref, dp_ref, g1_ref):
    dinv = _dinv_from_parts(dp_ref[0, :], dp_ref[1, :])
    g1_ref[...] = h_ref[...] * dinv[:, None]


def _tc2_body(acc_ref, g1_ref, dp_ref, w2_ref, b1_ref, g2_ref):
    dinv = _dinv_from_parts(dp_ref[0, :], dp_ref[1, :])
    t = acc_ref[0] + acc_ref[1] - g1_ref[...]
    pre = t * dinv[:, None] + b1_ref[...][None, :]
    r = jnp.maximum(pre, 0.0)
    h2 = jnp.dot(r, w2_ref[...], preferred_element_type=jnp.float32)
    g2_ref[...] = h2 * dinv[:, None]


def _tc3_body(acc_ref, g2_ref, dp_ref, b2_ref, out_ref):
    dinv = _dinv_from_parts(dp_ref[0, :], dp_ref[1, :])
    t = acc_ref[0] + acc_ref[1] - g2_ref[...]
    o = t * dinv[:, None] + b2_ref[...][None, :]
    m = jnp.max(o, axis=1, keepdims=True)
    e = jnp.exp(o - m)
    lse = jnp.log(jnp.sum(e, axis=1, keepdims=True))
    out_ref[...] = o - m - lse


def kernel(x, edge_index, edge_weight, W1, b1, W2, b2):
    n, d_in = x.shape
    e = edge_weight.shape[0]
    d_hid = W1.shape[1]
    d_out = W2.shape[1]
    epw = e // NW
    nch = epw // K

    src3 = edge_index[0].reshape(NW, nch, K)
    dst3 = edge_index[1].reshape(NW, nch, K)
    ew3 = edge_weight.reshape(NW, nch, K)

    h = _tc_call(_tc0_body, jax.ShapeDtypeStruct((n, d_hid), jnp.float32),
                 [x, W1])
    degp = _make_deg_kernel(n, nch)(dst3, ew3)
    g1 = _tc_call(_tc1_body, jax.ShapeDtypeStruct((n, d_hid), jnp.float32),
                  [h, degp])
    acc1 = _make_msg_kernel(n, nch, d_hid, 2, 1)(g1, src3, dst3, ew3)
    g2 = _tc_call(_tc2_body, jax.ShapeDtypeStruct((n, d_out), jnp.float32),
                  [acc1, g1, degp, W2, b1])
    acc2 = _make_msg_kernel(n, nch, d_out, 5, 0)(g2, src3, dst3, ew3)
    out = _tc_call(_tc3_body, jax.ShapeDtypeStruct((n, d_out), jnp.float32),
                   [acc2, g2, degp, b2])
    return out


# speedup vs baseline: 28.0245x; 1.0311x over previous
import jax
import jax.numpy as jnp
from jax import lax
from jax.experimental import pallas as pl
from jax.experimental.pallas import tpu as pltpu
from jax.experimental.pallas import tpu_sc as plsc
import functools

NC = 2
NS = 16
NW = NC * NS
K = 80


def _wid():
    c = lax.axis_index("c")
    s = lax.axis_index("s")
    return s * NC + c, c, s


def _make_deg_kernel(n, nch):
    mesh = plsc.VectorSubcoreMesh(core_axis_name="c", subcore_axis_name="s")

    @functools.partial(
        pl.kernel,
        out_type=jax.ShapeDtypeStruct((NC, n), jnp.float32),
        mesh=mesh,
        compiler_params=pltpu.CompilerParams(use_tc_tiling_on_sc=False, needs_layout_passes=False),
        scratch_types=[
            pltpu.VMEM((nch, K), jnp.int32),
            pltpu.VMEM((nch, K), jnp.float32),
            pltpu.VMEM((n,), jnp.float32),
            pltpu.SemaphoreType.DMA,
            pltpu.VMEM_SHARED((n,), jnp.float32),
        ],
    )
    def deg_kernel(dst_hbm, ew_hbm, out_hbm, dst_v, ew_v, tmp_v, sem, acc_sh):
        wid, c, s = _wid()
        pltpu.sync_copy(dst_hbm.at[wid], dst_v)
        pltpu.sync_copy(ew_hbm.at[wid], ew_v)

        @pl.when(s == 0)
        def _zero():
            def zrow(i, _):
                tmp_v[pl.ds(i * 16, 16)] = jnp.zeros((16,), jnp.float32)
                return 0
            lax.fori_loop(0, n // 16, zrow, 0)
            pltpu.sync_copy(tmp_v, acc_sh)

        plsc.subcore_barrier()

        def fire(j, _):
            pltpu.make_async_copy(ew_v.at[j], acc_sh.at[dst_v.at[j]],
                                  sem).start(add=True)
            return 0
        lax.fori_loop(0, nch, fire, 0)

        def drain(j, _):
            pltpu.make_async_copy(ew_v.at[j], acc_sh.at[dst_v.at[j]],
                                  sem).wait()
            return 0
        lax.fori_loop(0, nch, drain, 0)

        plsc.subcore_barrier()

        @pl.when(s == 0)
        def _writeout():
            pltpu.sync_copy(acc_sh, tmp_v)
            pltpu.sync_copy(tmp_v, out_hbm.at[c])

    return deg_kernel


def _make_msg_kernel(n, nch, d, nbuf, prologue):
    mesh = plsc.VectorSubcoreMesh(core_axis_name="c", subcore_axis_name="s")
    rows_per_sub = n // NS
    m = (nch - prologue) // nbuf
    nfc = d // 16

    @functools.partial(
        pl.kernel,
        out_type=jax.ShapeDtypeStruct((NC, n, d), jnp.float32),
        mesh=mesh,
        compiler_params=pltpu.CompilerParams(use_tc_tiling_on_sc=False, needs_layout_passes=False),
        scratch_types=[
            pltpu.VMEM((nch, K), jnp.int32),
            pltpu.VMEM((nch, K), jnp.int32),
            pltpu.VMEM((nch, K), jnp.float32),
        ] + [pltpu.VMEM((K, d), jnp.float32) for _ in range(nbuf)]
          + [pltpu.SemaphoreType.DMA for _ in range(2 * nbuf)]
          + [pltpu.VMEM_SHARED((n, d), jnp.float32)],
    )
    def msg_kernel(g_hbm, src_hbm, dst_hbm, ew_hbm, out_hbm,
                   src_v, dst_v, ew_v, *rest):
        rows = rest[:nbuf]
        sems = rest[nbuf:2 * nbuf]
        ssems = rest[2 * nbuf:3 * nbuf]
        acc_sh = rest[3 * nbuf]
        wid, c, s = _wid()
        pltpu.sync_copy(src_hbm.at[wid], src_v)
        pltpu.sync_copy(dst_hbm.at[wid], dst_v)
        pltpu.sync_copy(ew_hbm.at[wid], ew_v)

        base = s * rows_per_sub
        pltpu.sync_copy(g_hbm.at[pl.ds(base, rows_per_sub)],
                        acc_sh.at[pl.ds(base, rows_per_sub)])

        plsc.subcore_barrier()

        def scale(j, rv):
            def scale_grp(g2, _):
                for l in range(16):
                    i = g2 * 16 + l
                    spl = plsc.load_gather(
                        ew_v,
                        [jnp.full((16,), j, jnp.int32),
                         jnp.full((16,), i, jnp.int32)],
                    )
                    for fc in range(nfc):
                        rv[i, pl.ds(fc * 16, 16)] = (
                            rv[i, pl.ds(fc * 16, 16)] * spl)
                return 0
            lax.fori_loop(0, K // 16, scale_grp, 0)

        def start(j, b):
            pltpu.make_async_copy(g_hbm.at[src_v.at[j]], rows[b],
                                  sems[b]).start()

        def wait(j, b):
            pltpu.make_async_copy(g_hbm.at[src_v.at[j]], rows[b],
                                  sems[b]).wait()

        def start_scatter(j, b):
            pltpu.make_async_copy(rows[b], acc_sh.at[dst_v.at[j]],
                                  ssems[b]).start(add=True)

        def wait_scatter(j, b):
            pltpu.make_async_copy(rows[b], acc_sh.at[dst_v.at[j]],
                                  ssems[b]).wait()

        for p in range(prologue):
            pltpu.sync_copy(g_hbm.at[src_v.at[p]], rows[0])
            scale(p, rows[0])
            pltpu.sync_copy(rows[0], acc_sh.at[dst_v.at[p]], add=True)

        for b in range(nbuf):
            start(prologue + b, b)

        def grp(g, _):
            for b in range(nbuf):
                j = prologue + g * nbuf + b
                pb = (b - 1) % nbuf
                wait(j, b)

                @pl.when(j - 1 >= prologue)
                def _release():
                    wait_scatter(j - 1, pb)

                    @pl.when(j - 1 + nbuf < nch)
                    def _refill():
                        start(j - 1 + nbuf, pb)

                scale(j, rows[b])
                start_scatter(j, b)
            return 0
        lax.fori_loop(0, m, grp, 0)

        wait_scatter(nch - 1, nbuf - 1)

        plsc.subcore_barrier()

        pltpu.sync_copy(acc_sh.at[pl.ds(base, rows_per_sub)],
                        out_hbm.at[c].at[pl.ds(base, rows_per_sub)])

    return msg_kernel


def _tc_call(body, out_shapes, inputs):
    return pl.pallas_call(body, out_shape=out_shapes)(*inputs)


def _dinv_from_parts(dp0, dp1):
    deg = 1.0 + dp0 + dp1
    return jnp.where(deg > 0, lax.rsqrt(deg), 0.0)


def _tc0_body(x_ref, w1_ref, h_ref):
    h_ref[...] = jnp.dot(x_ref[...], w1_ref[...],
                         preferred_element_type=jnp.float32)


def _tc1_body(h_ref, dp_ref, g1_ref):
    dinv = _dinv_from_parts(dp_ref[0, :], dp_ref[1, :])
    g1_ref[...] = h_ref[...] * dinv[:, None]


def _tc2_body(acc_ref, g1_ref, dp_ref, w2_ref, b1_ref, g2_ref):
    dinv = _dinv_from_parts(dp_ref[0, :], dp_ref[1, :])
    t = acc_ref[0] + acc_ref[1] - g1_ref[...]
    pre = t * dinv[:, None] + b1_ref[...][None, :]
    r = jnp.maximum(pre, 0.0)
    h2 = jnp.dot(r, w2_ref[...], preferred_element_type=jnp.float32)
    g2_ref[...] = h2 * dinv[:, None]


def _tc3_body(acc_ref, g2_ref, dp_ref, b2_ref, out_ref):
    dinv = _dinv_from_parts(dp_ref[0, :], dp_ref[1, :])
    t = acc_ref[0] + acc_ref[1] - g2_ref[...]
    o = t * dinv[:, None] + b2_ref[...][None, :]
    m = jnp.max(o, axis=1, keepdims=True)
    e = jnp.exp(o - m)
    lse = jnp.log(jnp.sum(e, axis=1, keepdims=True))
    out_ref[...] = o - m - lse


def kernel(x, edge_index, edge_weight, W1, b1, W2, b2):
    n, d_in = x.shape
    e = edge_weight.shape[0]
    d_hid = W1.shape[1]
    d_out = W2.shape[1]
    epw = e // NW
    nch = epw // K

    src3 = edge_index[0].reshape(NW, nch, K)
    dst3 = edge_index[1].reshape(NW, nch, K)
    ew3 = edge_weight.reshape(NW, nch, K)

    h = _tc_call(_tc0_body, jax.ShapeDtypeStruct((n, d_hid), jnp.float32),
                 [x, W1])
    degp = _make_deg_kernel(n, nch)(dst3, ew3)
    g1 = _tc_call(_tc1_body, jax.ShapeDtypeStruct((n, d_hid), jnp.float32),
                  [h, degp])
    acc1 = _make_msg_kernel(n, nch, d_hid, 2, 1)(g1, src3, dst3, ew3)
    g2 = _tc_call(_tc2_body, jax.ShapeDtypeStruct((n, d_out), jnp.float32),
                  [acc1, g1, degp, W2, b1])
    acc2 = _make_msg_kernel(n, nch, d_out, 5, 0)(g2, src3, dst3, ew3)
    out = _tc_call(_tc3_body, jax.ShapeDtypeStruct((n, d_out), jnp.float32),
                   [acc2, g2, degp, b2])
    return out


# speedup vs baseline: 34.7281x; 1.2392x over previous
import jax
import jax.numpy as jnp
from jax import lax
from jax.experimental import pallas as pl
from jax.experimental.pallas import tpu as pltpu
from jax.experimental.pallas import tpu_sc as plsc
import functools

NC = 2
NS = 16
NW = NC * NS
K = 80


def _wid():
    c = lax.axis_index("c")
    s = lax.axis_index("s")
    return s * NC + c, c, s


def _make_deg_kernel(n, nch):
    mesh = plsc.VectorSubcoreMesh(core_axis_name="c", subcore_axis_name="s")

    @functools.partial(
        pl.kernel,
        out_type=jax.ShapeDtypeStruct((NC, n), jnp.float32),
        mesh=mesh,
        compiler_params=pltpu.CompilerParams(use_tc_tiling_on_sc=False, needs_layout_passes=False),
        scratch_types=[
            pltpu.VMEM((nch, K), jnp.int32),
            pltpu.VMEM((nch, K), jnp.float32),
            pltpu.VMEM((n,), jnp.float32),
            pltpu.SemaphoreType.DMA,
            pltpu.VMEM_SHARED((n,), jnp.float32),
        ],
    )
    def deg_kernel(dst_hbm, ew_hbm, out_hbm, dst_v, ew_v, tmp_v, sem, acc_sh):
        wid, c, s = _wid()
        pltpu.sync_copy(dst_hbm.at[wid], dst_v)
        pltpu.sync_copy(ew_hbm.at[wid], ew_v)

        @pl.when(s == 0)
        def _zero():
            def zrow(i, _):
                tmp_v[pl.ds(i * 16, 16)] = jnp.zeros((16,), jnp.float32)
                return 0
            lax.fori_loop(0, n // 16, zrow, 0)
            pltpu.sync_copy(tmp_v, acc_sh)

        plsc.subcore_barrier()

        def fire(j, _):
            pltpu.make_async_copy(ew_v.at[j], acc_sh.at[dst_v.at[j]],
                                  sem).start(add=True)
            return 0
        lax.fori_loop(0, nch, fire, 0)

        def drain(j, _):
            pltpu.make_async_copy(ew_v.at[j], acc_sh.at[dst_v.at[j]],
                                  sem).wait()
            return 0
        lax.fori_loop(0, nch, drain, 0)

        plsc.subcore_barrier()

        @pl.when(s == 0)
        def _writeout():
            pltpu.sync_copy(acc_sh, tmp_v)
            pltpu.sync_copy(tmp_v, out_hbm.at[c])

    return deg_kernel


def _make_msg_kernel(n, nch, d, nbuf, prologue):
    mesh = plsc.VectorSubcoreMesh(core_axis_name="c", subcore_axis_name="s")
    rows_per_sub = n // NS
    m = (nch - prologue) // nbuf
    nfc = d // 16

    @functools.partial(
        pl.kernel,
        out_type=jax.ShapeDtypeStruct((NC, n, d), jnp.float32),
        mesh=mesh,
        compiler_params=pltpu.CompilerParams(use_tc_tiling_on_sc=False, needs_layout_passes=False),
        scratch_types=[
            pltpu.VMEM((nch, K), jnp.int32),
            pltpu.VMEM((nch, K), jnp.int32),
            pltpu.VMEM((nch, K), jnp.float32),
        ] + [pltpu.VMEM((K, d), jnp.float32) for _ in range(nbuf)]
          + [pltpu.SemaphoreType.DMA for _ in range(2 * nbuf)]
          + [pltpu.VMEM_SHARED((n, d), jnp.float32)],
    )
    def msg_kernel(g_hbm, src_hbm, dst_hbm, ew_hbm, out_hbm,
                   src_v, dst_v, ew_v, *rest):
        rows = rest[:nbuf]
        sems = rest[nbuf:2 * nbuf]
        ssems = rest[2 * nbuf:3 * nbuf]
        acc_sh = rest[3 * nbuf]
        wid, c, s = _wid()
        pltpu.sync_copy(src_hbm.at[wid], src_v)
        pltpu.sync_copy(dst_hbm.at[wid], dst_v)
        pltpu.sync_copy(ew_hbm.at[wid], ew_v)

        base = s * rows_per_sub
        pltpu.sync_copy(g_hbm.at[pl.ds(base, rows_per_sub)],
                        acc_sh.at[pl.ds(base, rows_per_sub)])

        plsc.subcore_barrier()

        def scale(j, rv):
            @plsc.parallel_loop(0, K, unroll=8)
            def _row(i):
                spl = plsc.load_gather(
                    ew_v,
                    [jnp.full((16,), j, jnp.int32),
                     jnp.full((16,), i, jnp.int32)],
                )
                for fc in range(nfc):
                    rv[i, pl.ds(fc * 16, 16)] = (
                        rv[i, pl.ds(fc * 16, 16)] * spl)

        def start(j, b):
            pltpu.make_async_copy(g_hbm.at[src_v.at[j]], rows[b],
                                  sems[b]).start()

        def wait(j, b):
            pltpu.make_async_copy(g_hbm.at[src_v.at[j]], rows[b],
                                  sems[b]).wait()

        def start_scatter(j, b):
            pltpu.make_async_copy(rows[b], acc_sh.at[dst_v.at[j]],
                                  ssems[b]).start(add=True)

        def wait_scatter(j, b):
            pltpu.make_async_copy(rows[b], acc_sh.at[dst_v.at[j]],
                                  ssems[b]).wait()

        for p in range(prologue):
            pltpu.sync_copy(g_hbm.at[src_v.at[p]], rows[0])
            scale(p, rows[0])
            pltpu.sync_copy(rows[0], acc_sh.at[dst_v.at[p]], add=True)

        for b in range(nbuf):
            start(prologue + b, b)

        def grp(g, _):
            for b in range(nbuf):
                j = prologue + g * nbuf + b
                pb = (b - 1) % nbuf
                wait(j, b)

                @pl.when(j - 1 >= prologue)
                def _release():
                    wait_scatter(j - 1, pb)

                    @pl.when(j - 1 + nbuf < nch)
                    def _refill():
                        start(j - 1 + nbuf, pb)

                scale(j, rows[b])
                start_scatter(j, b)
            return 0
        lax.fori_loop(0, m, grp, 0)

        wait_scatter(nch - 1, nbuf - 1)

        plsc.subcore_barrier()

        pltpu.sync_copy(acc_sh.at[pl.ds(base, rows_per_sub)],
                        out_hbm.at[c].at[pl.ds(base, rows_per_sub)])

    return msg_kernel


def _tc_call(body, out_shapes, inputs):
    return pl.pallas_call(body, out_shape=out_shapes)(*inputs)


def _dinv_from_parts(dp0, dp1):
    deg = 1.0 + dp0 + dp1
    return jnp.where(deg > 0, lax.rsqrt(deg), 0.0)


def _tc0_body(x_ref, w1_ref, h_ref):
    h_ref[...] = jnp.dot(x_ref[...], w1_ref[...],
                         preferred_element_type=jnp.float32)


def _tc1_body(h_ref, dp_ref, g1_ref):
    dinv = _dinv_from_parts(dp_ref[0, :], dp_ref[1, :])
    g1_ref[...] = h_ref[...] * dinv[:, None]


def _tc2_body(acc_ref, g1_ref, dp_ref, w2_ref, b1_ref, g2_ref):
    dinv = _dinv_from_parts(dp_ref[0, :], dp_ref[1, :])
    t = acc_ref[0] + acc_ref[1] - g1_ref[...]
    pre = t * dinv[:, None] + b1_ref[...][None, :]
    r = jnp.maximum(pre, 0.0)
    h2 = jnp.dot(r, w2_ref[...], preferred_element_type=jnp.float32)
    g2_ref[...] = h2 * dinv[:, None]


def _tc3_body(acc_ref, g2_ref, dp_ref, b2_ref, out_ref):
    dinv = _dinv_from_parts(dp_ref[0, :], dp_ref[1, :])
    t = acc_ref[0] + acc_ref[1] - g2_ref[...]
    o = t * dinv[:, None] + b2_ref[...][None, :]
    m = jnp.max(o, axis=1, keepdims=True)
    e = jnp.exp(o - m)
    lse = jnp.log(jnp.sum(e, axis=1, keepdims=True))
    out_ref[...] = o - m - lse


def kernel(x, edge_index, edge_weight, W1, b1, W2, b2):
    n, d_in = x.shape
    e = edge_weight.shape[0]
    d_hid = W1.shape[1]
    d_out = W2.shape[1]
    epw = e // NW
    nch = epw // K

    src3 = edge_index[0].reshape(NW, nch, K)
    dst3 = edge_index[1].reshape(NW, nch, K)
    ew3 = edge_weight.reshape(NW, nch, K)

    h = _tc_call(_tc0_body, jax.ShapeDtypeStruct((n, d_hid), jnp.float32),
                 [x, W1])
    degp = _make_deg_kernel(n, nch)(dst3, ew3)
    g1 = _tc_call(_tc1_body, jax.ShapeDtypeStruct((n, d_hid), jnp.float32),
                  [h, degp])
    acc1 = _make_msg_kernel(n, nch, d_hid, 2, 1)(g1, src3, dst3, ew3)
    g2 = _tc_call(_tc2_body, jax.ShapeDtypeStruct((n, d_out), jnp.float32),
                  [acc1, g1, degp, W2, b1])
    acc2 = _make_msg_kernel(n, nch, d_out, 5, 0)(g2, src3, dst3, ew3)
    out = _tc_call(_tc3_body, jax.ShapeDtypeStruct((n, d_out), jnp.float32),
                   [acc2, g2, degp, b2])
    return out


# speedup vs baseline: 37.7557x; 1.0872x over previous
import jax
import jax.numpy as jnp
from jax import lax
from jax.experimental import pallas as pl
from jax.experimental.pallas import tpu as pltpu
from jax.experimental.pallas import tpu_sc as plsc
import functools

NC = 2
NS = 16
NW = NC * NS
K = 80


def _wid():
    c = lax.axis_index("c")
    s = lax.axis_index("s")
    return s * NC + c, c, s


def _make_deg_kernel(n, nch):
    mesh = plsc.VectorSubcoreMesh(core_axis_name="c", subcore_axis_name="s")

    @functools.partial(
        pl.kernel,
        out_type=jax.ShapeDtypeStruct((NC, n), jnp.float32),
        mesh=mesh,
        compiler_params=pltpu.CompilerParams(use_tc_tiling_on_sc=False, needs_layout_passes=False),
        scratch_types=[
            pltpu.VMEM((nch, K), jnp.int32),
            pltpu.VMEM((nch, K), jnp.float32),
            pltpu.VMEM((n,), jnp.float32),
            pltpu.SemaphoreType.DMA,
            pltpu.VMEM_SHARED((n,), jnp.float32),
        ],
    )
    def deg_kernel(dst_hbm, ew_hbm, out_hbm, dst_v, ew_v, tmp_v, sem, acc_sh):
        wid, c, s = _wid()
        pltpu.sync_copy(dst_hbm.at[wid], dst_v)
        pltpu.sync_copy(ew_hbm.at[wid], ew_v)

        @pl.when(s == 0)
        def _zero():
            def zrow(i, _):
                tmp_v[pl.ds(i * 16, 16)] = jnp.zeros((16,), jnp.float32)
                return 0
            lax.fori_loop(0, n // 16, zrow, 0)
            pltpu.sync_copy(tmp_v, acc_sh)

        plsc.subcore_barrier()

        def fire(j, _):
            pltpu.make_async_copy(ew_v.at[j], acc_sh.at[dst_v.at[j]],
                                  sem).start(add=True)
            return 0
        lax.fori_loop(0, nch, fire, 0)

        def drain(j, _):
            pltpu.make_async_copy(ew_v.at[j], acc_sh.at[dst_v.at[j]],
                                  sem).wait()
            return 0
        lax.fori_loop(0, nch, drain, 0)

        plsc.subcore_barrier()

        @pl.when(s == 0)
        def _writeout():
            pltpu.sync_copy(acc_sh, tmp_v)
            pltpu.sync_copy(tmp_v, out_hbm.at[c])

    return deg_kernel


def _make_msg_kernel(n, nch, d, nbuf, prologue, k=K):
    mesh = plsc.VectorSubcoreMesh(core_axis_name="c", subcore_axis_name="s")
    rows_per_sub = n // NS
    m = (nch - prologue) // nbuf
    nfc = d // 16

    @functools.partial(
        pl.kernel,
        out_type=jax.ShapeDtypeStruct((NC, n, d), jnp.float32),
        mesh=mesh,
        compiler_params=pltpu.CompilerParams(use_tc_tiling_on_sc=False, needs_layout_passes=False),
        scratch_types=[
            pltpu.VMEM((nch, k), jnp.int32),
            pltpu.VMEM((nch, k), jnp.int32),
            pltpu.VMEM((nch, k), jnp.float32),
        ] + [pltpu.VMEM((k, d), jnp.float32) for _ in range(nbuf)]
          + [pltpu.SemaphoreType.DMA for _ in range(2 * nbuf)]
          + [pltpu.VMEM_SHARED((n, d), jnp.float32)],
    )
    def msg_kernel(g_hbm, src_hbm, dst_hbm, ew_hbm, out_hbm,
                   src_v, dst_v, ew_v, *rest):
        rows = rest[:nbuf]
        sems = rest[nbuf:2 * nbuf]
        ssems = rest[2 * nbuf:3 * nbuf]
        acc_sh = rest[3 * nbuf]
        wid, c, s = _wid()
        pltpu.sync_copy(src_hbm.at[wid], src_v)
        pltpu.sync_copy(dst_hbm.at[wid], dst_v)
        pltpu.sync_copy(ew_hbm.at[wid], ew_v)

        base = s * rows_per_sub
        pltpu.sync_copy(g_hbm.at[pl.ds(base, rows_per_sub)],
                        acc_sh.at[pl.ds(base, rows_per_sub)])

        plsc.subcore_barrier()

        def scale(j, rv):
            @plsc.parallel_loop(0, k, unroll=8)
            def _row(i):
                spl = plsc.load_gather(
                    ew_v,
                    [jnp.full((16,), j, jnp.int32),
                     jnp.full((16,), i, jnp.int32)],
                )
                for fc in range(nfc):
                    rv[i, pl.ds(fc * 16, 16)] = (
                        rv[i, pl.ds(fc * 16, 16)] * spl)

        def start(j, b):
            pltpu.make_async_copy(g_hbm.at[src_v.at[j]], rows[b],
                                  sems[b]).start()

        def wait(j, b):
            pltpu.make_async_copy(g_hbm.at[src_v.at[j]], rows[b],
                                  sems[b]).wait()

        def start_scatter(j, b):
            pltpu.make_async_copy(rows[b], acc_sh.at[dst_v.at[j]],
                                  ssems[b]).start(add=True)

        def wait_scatter(j, b):
            pltpu.make_async_copy(rows[b], acc_sh.at[dst_v.at[j]],
                                  ssems[b]).wait()

        for p in range(prologue):
            pltpu.sync_copy(g_hbm.at[src_v.at[p]], rows[0])
            scale(p, rows[0])
            pltpu.sync_copy(rows[0], acc_sh.at[dst_v.at[p]], add=True)

        for b in range(nbuf):
            start(prologue + b, b)

        def grp(g, _):
            for b in range(nbuf):
                j = prologue + g * nbuf + b
                pb = (b - 1) % nbuf
                wait(j, b)

                @pl.when(j - 1 >= prologue)
                def _release():
                    wait_scatter(j - 1, pb)

                    @pl.when(j - 1 + nbuf < nch)
                    def _refill():
                        start(j - 1 + nbuf, pb)

                scale(j, rows[b])
                start_scatter(j, b)
            return 0
        lax.fori_loop(0, m, grp, 0)

        wait_scatter(nch - 1, nbuf - 1)

        plsc.subcore_barrier()

        pltpu.sync_copy(acc_sh.at[pl.ds(base, rows_per_sub)],
                        out_hbm.at[c].at[pl.ds(base, rows_per_sub)])

    return msg_kernel


def _tc_call(body, out_shapes, inputs):
    return pl.pallas_call(body, out_shape=out_shapes)(*inputs)


def _dinv_from_parts(dp0, dp1):
    deg = 1.0 + dp0 + dp1
    return jnp.where(deg > 0, lax.rsqrt(deg), 0.0)


def _tc0_body(x_ref, w1_ref, h_ref):
    h_ref[...] = jnp.dot(x_ref[...], w1_ref[...],
                         preferred_element_type=jnp.float32)


def _tc1_body(h_ref, dp_ref, g1_ref):
    dinv = _dinv_from_parts(dp_ref[0, :], dp_ref[1, :])
    g1_ref[...] = h_ref[...] * dinv[:, None]


def _tc2_body(acc_ref, g1_ref, dp_ref, w2_ref, b1_ref, g2_ref):
    dinv = _dinv_from_parts(dp_ref[0, :], dp_ref[1, :])
    t = acc_ref[0] + acc_ref[1] - g1_ref[...]
    pre = t * dinv[:, None] + b1_ref[...][None, :]
    r = jnp.maximum(pre, 0.0)
    h2 = jnp.dot(r, w2_ref[...], preferred_element_type=jnp.float32)
    g2_ref[...] = h2 * dinv[:, None]


def _tc3_body(acc_ref, g2_ref, dp_ref, b2_ref, out_ref):
    dinv = _dinv_from_parts(dp_ref[0, :], dp_ref[1, :])
    t = acc_ref[0] + acc_ref[1] - g2_ref[...]
    o = t * dinv[:, None] + b2_ref[...][None, :]
    m = jnp.max(o, axis=1, keepdims=True)
    e = jnp.exp(o - m)
    lse = jnp.log(jnp.sum(e, axis=1, keepdims=True))
    out_ref[...] = o - m - lse


def kernel(x, edge_index, edge_weight, W1, b1, W2, b2):
    n, d_in = x.shape
    e = edge_weight.shape[0]
    d_hid = W1.shape[1]
    d_out = W2.shape[1]
    epw = e // NW
    nch = epw // K

    src3 = edge_index[0].reshape(NW, nch, K)
    dst3 = edge_index[1].reshape(NW, nch, K)
    ew3 = edge_weight.reshape(NW, nch, K)

    k1m = 40
    nch1 = epw // k1m
    src3a = edge_index[0].reshape(NW, nch1, k1m)
    dst3a = edge_index[1].reshape(NW, nch1, k1m)
    ew3a = edge_weight.reshape(NW, nch1, k1m)

    h = _tc_call(_tc0_body, jax.ShapeDtypeStruct((n, d_hid), jnp.float32),
                 [x, W1])
    degp = _make_deg_kernel(n, nch)(dst3, ew3)
    g1 = _tc_call(_tc1_body, jax.ShapeDtypeStruct((n, d_hid), jnp.float32),
                  [h, degp])
    acc1 = _make_msg_kernel(n, nch1, d_hid, 4, 2, k1m)(g1, src3a, dst3a, ew3a)
    g2 = _tc_call(_tc2_body, jax.ShapeDtypeStruct((n, d_out), jnp.float32),
                  [acc1, g1, degp, W2, b1])
    acc2 = _make_msg_kernel(n, nch, d_out, 5, 0)(g2, src3, dst3, ew3)
    out = _tc_call(_tc3_body, jax.ShapeDtypeStruct((n, d_out), jnp.float32),
                   [acc2, g2, degp, b2])
    return out


# speedup vs baseline: 41.0159x; 1.0863x over previous
import jax
import jax.numpy as jnp
from jax import lax
from jax.experimental import pallas as pl
from jax.experimental.pallas import tpu as pltpu
from jax.experimental.pallas import tpu_sc as plsc
import functools

NC = 2
NS = 16
NW = NC * NS
K = 80


def _wid():
    c = lax.axis_index("c")
    s = lax.axis_index("s")
    return s * NC + c, c, s


def _make_deg_kernel(n, nch):
    mesh = plsc.VectorSubcoreMesh(core_axis_name="c", subcore_axis_name="s")

    @functools.partial(
        pl.kernel,
        out_type=jax.ShapeDtypeStruct((NC, n), jnp.float32),
        mesh=mesh,
        compiler_params=pltpu.CompilerParams(use_tc_tiling_on_sc=False, needs_layout_passes=False),
        scratch_types=[
            pltpu.VMEM((nch, K), jnp.int32),
            pltpu.VMEM((nch, K), jnp.float32),
            pltpu.VMEM((n,), jnp.float32),
            pltpu.SemaphoreType.DMA,
            pltpu.VMEM_SHARED((n,), jnp.float32),
        ],
    )
    def deg_kernel(dst_hbm, ew_hbm, out_hbm, dst_v, ew_v, tmp_v, sem, acc_sh):
        wid, c, s = _wid()
        pltpu.sync_copy(dst_hbm.at[wid], dst_v)
        pltpu.sync_copy(ew_hbm.at[wid], ew_v)

        @pl.when(s == 0)
        def _zero():
            def zrow(i, _):
                tmp_v[pl.ds(i * 16, 16)] = jnp.zeros((16,), jnp.float32)
                return 0
            lax.fori_loop(0, n // 16, zrow, 0)
            pltpu.sync_copy(tmp_v, acc_sh)

        plsc.subcore_barrier()

        def fire(j, _):
            pltpu.make_async_copy(ew_v.at[j], acc_sh.at[dst_v.at[j]],
                                  sem).start(add=True)
            return 0
        lax.fori_loop(0, nch, fire, 0)

        def drain(j, _):
            pltpu.make_async_copy(ew_v.at[j], acc_sh.at[dst_v.at[j]],
                                  sem).wait()
            return 0
        lax.fori_loop(0, nch, drain, 0)

        plsc.subcore_barrier()

        @pl.when(s == 0)
        def _writeout():
            pltpu.sync_copy(acc_sh, tmp_v)
            pltpu.sync_copy(tmp_v, out_hbm.at[c])

    return deg_kernel


def _make_msg_kernel(n, nch, d, nbuf, prologue, k=K):
    mesh = plsc.VectorSubcoreMesh(core_axis_name="c", subcore_axis_name="s")
    rows_per_sub = n // NS
    m = (nch - prologue) // nbuf
    nfc = d // 16

    @functools.partial(
        pl.kernel,
        out_type=jax.ShapeDtypeStruct((NC, n, d), jnp.float32),
        mesh=mesh,
        compiler_params=pltpu.CompilerParams(use_tc_tiling_on_sc=False, needs_layout_passes=False),
        scratch_types=[
            pltpu.VMEM((nch, k), jnp.int32),
            pltpu.VMEM((nch, k), jnp.int32),
            pltpu.VMEM((nch, k), jnp.float32),
        ] + [pltpu.VMEM((k, d), jnp.float32) for _ in range(nbuf)]
          + [pltpu.SemaphoreType.DMA for _ in range(2 * nbuf)]
          + [pltpu.VMEM_SHARED((n, d), jnp.float32)],
    )
    def msg_kernel(g_hbm, src_hbm, dst_hbm, ew_hbm, out_hbm,
                   src_v, dst_v, ew_v, *rest):
        rows = rest[:nbuf]
        sems = rest[nbuf:2 * nbuf]
        ssems = rest[2 * nbuf:3 * nbuf]
        acc_sh = rest[3 * nbuf]
        wid, c, s = _wid()
        pltpu.sync_copy(src_hbm.at[wid], src_v)
        pltpu.sync_copy(dst_hbm.at[wid], dst_v)
        pltpu.sync_copy(ew_hbm.at[wid], ew_v)

        base = s * rows_per_sub
        pltpu.sync_copy(g_hbm.at[pl.ds(base, rows_per_sub)],
                        acc_sh.at[pl.ds(base, rows_per_sub)])

        plsc.subcore_barrier()

        def scale(j, rv):
            @plsc.parallel_loop(0, k, unroll=8)
            def _row(i):
                spl = plsc.load_gather(
                    ew_v,
                    [jnp.full((16,), j, jnp.int32),
                     jnp.full((16,), i, jnp.int32)],
                )
                for fc in range(nfc):
                    rv[i, pl.ds(fc * 16, 16)] = (
                        rv[i, pl.ds(fc * 16, 16)] * spl)

        def start(j, b):
            pltpu.make_async_copy(g_hbm.at[src_v.at[j]], rows[b],
                                  sems[b]).start()

        def wait(j, b):
            pltpu.make_async_copy(g_hbm.at[src_v.at[j]], rows[b],
                                  sems[b]).wait()

        def start_scatter(j, b):
            pltpu.make_async_copy(rows[b], acc_sh.at[dst_v.at[j]],
                                  ssems[b]).start(add=True)

        def wait_scatter(j, b):
            pltpu.make_async_copy(rows[b], acc_sh.at[dst_v.at[j]],
                                  ssems[b]).wait()

        for p in range(prologue):
            pltpu.sync_copy(g_hbm.at[src_v.at[p]], rows[0])
            scale(p, rows[0])
            pltpu.sync_copy(rows[0], acc_sh.at[dst_v.at[p]], add=True)

        for b in range(nbuf):
            start(prologue + b, b)

        def grp(g, _):
            for b in range(nbuf):
                j = prologue + g * nbuf + b
                pb = (b - 1) % nbuf
                wait(j, b)

                @pl.when(j - 1 >= prologue)
                def _release():
                    wait_scatter(j - 1, pb)

                    @pl.when(j - 1 + nbuf < nch)
                    def _refill():
                        start(j - 1 + nbuf, pb)

                scale(j, rows[b])
                start_scatter(j, b)
            return 0
        lax.fori_loop(0, m, grp, 0)

        wait_scatter(nch - 1, nbuf - 1)

        plsc.subcore_barrier()

        pltpu.sync_copy(acc_sh.at[pl.ds(base, rows_per_sub)],
                        out_hbm.at[c].at[pl.ds(base, rows_per_sub)])

    return msg_kernel


def _make_msg_kernel_bf16(n, nch, k):
    mesh = plsc.VectorSubcoreMesh(core_axis_name="c", subcore_axis_name="s")
    d = 128
    nbuf = 4
    prologue = 2
    rows_per_sub = n // NS
    m = (nch - prologue) // nbuf

    @functools.partial(
        pl.kernel,
        out_type=jax.ShapeDtypeStruct((NC, n, d), jnp.float32),
        mesh=mesh,
        compiler_params=pltpu.CompilerParams(use_tc_tiling_on_sc=False, needs_layout_passes=False),
        scratch_types=[
            pltpu.VMEM((nch, k), jnp.int32),
            pltpu.VMEM((nch, k), jnp.int32),
            pltpu.VMEM((nch, k), jnp.float32),
        ] + [pltpu.VMEM((k, d), jnp.bfloat16) for _ in range(nbuf)]
          + [pltpu.VMEM((k, d), jnp.float32) for _ in range(2)]
          + [pltpu.SemaphoreType.DMA for _ in range(nbuf + 2)]
          + [pltpu.VMEM_SHARED((n, d), jnp.float32)],
    )
    def msg_kernel(gb_hbm, gf_hbm, src_hbm, dst_hbm, ew_hbm, out_hbm,
                   src_v, dst_v, ew_v, *rest):
        gbufs = rest[:nbuf]
        sbufs = rest[nbuf:nbuf + 2]
        gsems = rest[nbuf + 2:2 * nbuf + 2]
        ssems = rest[2 * nbuf + 2:2 * nbuf + 4]
        acc_sh = rest[2 * nbuf + 4]
        wid, c, s = _wid()
        pltpu.sync_copy(src_hbm.at[wid], src_v)
        pltpu.sync_copy(dst_hbm.at[wid], dst_v)
        pltpu.sync_copy(ew_hbm.at[wid], ew_v)

        base = s * rows_per_sub
        pltpu.sync_copy(gf_hbm.at[pl.ds(base, rows_per_sub)],
                        acc_sh.at[pl.ds(base, rows_per_sub)])

        plsc.subcore_barrier()

        mask_hi = jnp.full((16,), 0xFFFF0000, jnp.uint32)

        def scale(j, gb, sb):
            @plsc.parallel_loop(0, k, unroll=8)
            def _row(i):
                spl = plsc.load_gather(
                    ew_v,
                    [jnp.full((16,), j, jnp.int32),
                     jnp.full((16,), i, jnp.int32)],
                )
                for c2 in range(d // 32):
                    vu = plsc.bitcast(gb[i, pl.ds(c2 * 32, 32)], jnp.uint32)
                    ev = plsc.bitcast(vu << 16, jnp.float32)
                    od = plsc.bitcast(vu & mask_hi, jnp.float32)
                    sb[i, pl.ds(c2 * 32, 16)] = ev * spl
                    sb[i, pl.ds(c2 * 32 + 16, 16)] = od * spl

        def start_g(j, b):
            pltpu.make_async_copy(gb_hbm.at[src_v.at[j]], gbufs[b],
                                  gsems[b]).start()

        def wait_g(j, b):
            pltpu.make_async_copy(gb_hbm.at[src_v.at[j]], gbufs[b],
                                  gsems[b]).wait()

        def start_s(j, sb):
            pltpu.make_async_copy(sbufs[sb], acc_sh.at[dst_v.at[j]],
                                  ssems[sb]).start(add=True)

        def wait_s(j, sb):
            pltpu.make_async_copy(sbufs[sb], acc_sh.at[dst_v.at[j]],
                                  ssems[sb]).wait()

        for p in range(prologue):
            pltpu.sync_copy(gb_hbm.at[src_v.at[p]], gbufs[0])
            scale(p, gbufs[0], sbufs[0])
            pltpu.sync_copy(sbufs[0], acc_sh.at[dst_v.at[p]], add=True)

        for b in range(nbuf):
            start_g(prologue + b, b)

        def grp(g, _):
            for b in range(nbuf):
                j = prologue + g * nbuf + b
                sb = b % 2
                wait_g(j, b)

                @pl.when(j >= prologue + 2)
                def _drain():
                    wait_s(j - 2, sb)

                scale(j, gbufs[b], sbufs[sb])
                start_s(j, sb)

                @pl.when(j + nbuf < nch)
                def _refill():
                    start_g(j + nbuf, b)
            return 0
        lax.fori_loop(0, m, grp, 0)

        wait_s(nch - 2, (nch - 2) % 2)
        wait_s(nch - 1, (nch - 1) % 2)

        plsc.subcore_barrier()

        pltpu.sync_copy(acc_sh.at[pl.ds(base, rows_per_sub)],
                        out_hbm.at[c].at[pl.ds(base, rows_per_sub)])

    return msg_kernel


def _tc_call(body, out_shapes, inputs):
    return pl.pallas_call(body, out_shape=out_shapes)(*inputs)


def _dinv_from_parts(dp0, dp1):
    deg = 1.0 + dp0 + dp1
    return jnp.where(deg > 0, lax.rsqrt(deg), 0.0)


def _tc0_body(x_ref, w1_ref, h_ref):
    h_ref[...] = jnp.dot(x_ref[...], w1_ref[...],
                         preferred_element_type=jnp.float32)


def _tc1_body(h_ref, dp_ref, g1b_ref, g1f_ref):
    dinv = _dinv_from_parts(dp_ref[0, :], dp_ref[1, :])
    g = h_ref[...] * dinv[:, None]
    g1b_ref[...] = g[:, :g1b_ref.shape[1]].astype(jnp.bfloat16)
    g1f_ref[...] = g[:, g1b_ref.shape[1]:]


def _tc2_body(acc_ref, g1_ref, dp_ref, w2_ref, b1_ref, g2_ref):
    dinv = _dinv_from_parts(dp_ref[0, :], dp_ref[1, :])
    t = acc_ref[0] + acc_ref[1] - g1_ref[...]
    pre = t * dinv[:, None] + b1_ref[...][None, :]
    r = jnp.maximum(pre, 0.0)
    h2 = jnp.dot(r, w2_ref[...], preferred_element_type=jnp.float32)
    g2_ref[...] = h2 * dinv[:, None]


def _tc3_body(acc_ref, g2_ref, dp_ref, b2_ref, out_ref):
    dinv = _dinv_from_parts(dp_ref[0, :], dp_ref[1, :])
    t = acc_ref[0] + acc_ref[1] - g2_ref[...]
    o = t * dinv[:, None] + b2_ref[...][None, :]
    m = jnp.max(o, axis=1, keepdims=True)
    e = jnp.exp(o - m)
    lse = jnp.log(jnp.sum(e, axis=1, keepdims=True))
    out_ref[...] = o - m - lse


def kernel(x, edge_index, edge_weight, W1, b1, W2, b2):
    n, d_in = x.shape
    e = edge_weight.shape[0]
    d_hid = W1.shape[1]
    d_out = W2.shape[1]
    epw = e // NW
    nch = epw // K

    src3 = edge_index[0].reshape(NW, nch, K)
    dst3 = edge_index[1].reshape(NW, nch, K)
    ew3 = edge_weight.reshape(NW, nch, K)

    k1m = 40
    nch1 = epw // k1m
    src3a = edge_index[0].reshape(NW, nch1, k1m)
    dst3a = edge_index[1].reshape(NW, nch1, k1m)
    ew3a = edge_weight.reshape(NW, nch1, k1m)

    blk = jnp.arange(0, d_hid, 32)[:, None]
    u = jnp.arange(16)[None, :]
    p0 = jnp.concatenate([blk + 2 * u, blk + 2 * u + 1], axis=1).reshape(-1)
    w1cat = jnp.concatenate([W1, W1[:, p0]], axis=1)
    w2p = W2[p0, :]
    b1p = b1[p0]

    h = _tc_call(_tc0_body, jax.ShapeDtypeStruct((n, 2 * d_hid), jnp.float32),
                 [x, w1cat])
    degp = _make_deg_kernel(n, nch)(dst3, ew3)
    g1b, g1f = _tc_call(
        _tc1_body,
        [jax.ShapeDtypeStruct((n, d_hid), jnp.bfloat16),
         jax.ShapeDtypeStruct((n, d_hid), jnp.float32)],
        [h, degp])
    acc1 = _make_msg_kernel_bf16(n, nch1, k1m)(g1b, g1f, src3a, dst3a, ew3a)
    g2 = _tc_call(_tc2_body, jax.ShapeDtypeStruct((n, d_out), jnp.float32),
                  [acc1, g1f, degp, w2p, b1p])
    acc2 = _make_msg_kernel(n, nch, d_out, 5, 0)(g2, src3, dst3, ew3)
    out = _tc_call(_tc3_body, jax.ShapeDtypeStruct((n, d_out), jnp.float32),
                   [acc2, g2, degp, b2])
    return out
